# baseline (speedup 1.0000x reference)
import functools

import jax
import jax.numpy as jnp
from jax import lax
from jax.experimental import pallas as pl
from jax.experimental.pallas import tpu as pltpu
from jax.experimental.pallas import tpu_sc as plsc

VOCAB = 1000000
EMBED_DIM = 32
BATCH = 4096
SEQ = 200

LANES = 16
SUB = 128
N_BT = BATCH // SUB
N_GROUPS = SEQ * N_BT
NW = 32
PER_W = N_GROUPS // NW
G = 4
N_CHUNKS = PER_W // G

N_WIN = (VOCAB - 64) // SUB
WIN_FLOOR = N_WIN // NW
WIN_EXTRA = N_WIN % NW
R_ROWS = VOCAB * EMBED_DIM // SUB

_mesh = plsc.VectorSubcoreMesh(core_axis_name="c", subcore_axis_name="s")


def _sum_body(wt_t, wn_t, tail, rs, at0, at1, an0, an1, o0, o1, tl_v,
              si0, si1, so0, so1):
    wid = lax.axis_index("s") * 2 + lax.axis_index("c")
    lane_iota = lax.iota(jnp.int32, LANES)

    at_v = (at0, at1)
    an_v = (an0, an1)
    o_v = (o0, o1)
    sem_i = (si0, si1)
    sem_o = (so0, so1)

    @pl.when(wid == 0)
    def _tail():
        pltpu.sync_copy(tail, tl_v)
        pltpu.sync_copy(tl_v, rs.at[pl.ds(R_ROWS - 16, 16)])

    def fire_in(jj, b):
        j = wid + NW * jj
        for db in range(4):
            pltpu.async_copy(
                wt_t.at[pl.ds(8 * db, 8), pl.ds(SUB * j, SUB)],
                at_v[b].at[db], sem_i[b])
            pltpu.async_copy(
                wn_t.at[pl.ds(8 * db, 8), pl.ds(SUB * j, SUB)],
                an_v[b].at[db], sem_i[b])

    def drain_in(b):
        for db in range(4):
            pltpu.make_async_copy(
                wt_t.at[pl.ds(0, 8), pl.ds(0, SUB)], at_v[b].at[db],
                sem_i[b]).wait()
            pltpu.make_async_copy(
                wn_t.at[pl.ds(0, 8), pl.ds(0, SUB)], an_v[b].at[db],
                sem_i[b]).wait()

    def drain_out(b):
        pltpu.make_async_copy(rs.at[pl.ds(0, 32)], o_v[b], sem_o[b]).wait()

    def sum_permute(b):
        def d_step(d0, _):
            dv = lax.rem(d0 + lane_iota, EMBED_DIM)
            dbv = dv // 8
            slv = dv - dbv * 8
            base = lane_iota * EMBED_DIM + dv
            qb = base // SUB
            cv2 = base - qb * SUB
            for k in range(8):
                cv = lane_iota + 16 * k
                x = (plsc.load_gather(at_v[b], [dbv, slv, cv]) +
                     plsc.load_gather(an_v[b], [dbv, slv, cv]))
                plsc.store_scatter(o_v[b], [qb + 4 * k, cv2], x)
            return _

        lax.fori_loop(0, EMBED_DIM, d_step, 0)

    fire_in(0, 0)
    fire_in(1, 1)

    def super_step(ss, _):
        for b in range(2):
            jj = ss * 2 + b
            drain_in(b)

            @pl.when(ss >= 1)
            def _do():
                drain_out(b)

            sum_permute(b)
            j = wid + NW * jj
            pltpu.async_copy(o_v[b], rs.at[pl.ds(32 * j, 32)], sem_o[b])

            @pl.when(ss < WIN_FLOOR // 2 - 1)
            def _pf():
                fire_in(jj + 2, b)

        return _

    lax.fori_loop(0, WIN_FLOOR // 2, super_step, 0)

    @pl.when(wid < WIN_EXTRA)
    def _extra():
        drain_out(0)
        j = wid + NW * WIN_FLOOR
        for db in range(4):
            pltpu.sync_copy(
                wt_t.at[pl.ds(8 * db, 8), pl.ds(SUB * j, SUB)],
                at_v[0].at[db])
            pltpu.sync_copy(
                wn_t.at[pl.ds(8 * db, 8), pl.ds(SUB * j, SUB)],
                an_v[0].at[db])
        sum_permute(0)
        pltpu.async_copy(o_v[0], rs.at[pl.ds(32 * j, 32)], sem_o[0])

    drain_out(0)
    drain_out(1)


_sumtable = functools.partial(
    pl.kernel,
    out_type=jax.ShapeDtypeStruct((R_ROWS, SUB), jnp.float32),
    mesh=_mesh,
    scratch_types=[
        pltpu.VMEM((4, 8, SUB), jnp.float32),
        pltpu.VMEM((4, 8, SUB), jnp.float32),
        pltpu.VMEM((4, 8, SUB), jnp.float32),
        pltpu.VMEM((4, 8, SUB), jnp.float32),
        pltpu.VMEM((32, SUB), jnp.float32),
        pltpu.VMEM((32, SUB), jnp.float32),
        pltpu.VMEM((16, SUB), jnp.float32),
        pltpu.SemaphoreType.DMA,
        pltpu.SemaphoreType.DMA,
        pltpu.SemaphoreType.DMA,
        pltpu.SemaphoreType.DMA,
    ],
    compiler_params=pltpu.CompilerParams(
        use_tc_tiling_on_sc=True, needs_layout_passes=False),
)(_sum_body)


def _body(ids_hbm, ws_hbm, out_hbm,
          idx0, idx1, t0, t1, tr0, tr1,
          sg0, sg1, so0, so1):
    wid = lax.axis_index("s") * 2 + lax.axis_index("c")
    row0_w = wid * PER_W
    lane_iota = lax.iota(jnp.int32, LANES)

    idx_v = (idx0, idx1)
    t_v = (t0, t1)
    tr_v = (tr0, tr1)
    sem_g = (sg0, sg1)
    sem_o = (so0, so1)

    def fire(c, b):
        pltpu.sync_copy(ids_hbm.at[pl.ds(row0_w + c * G, G)], idx_v[b])
        for g in range(G):
            pltpu.async_copy(
                ws_hbm.at[idx_v[b].at[g]], t_v[b].at[pl.ds(g * SUB, SUB)],
                sem_g[b])

    def drain_gathers(b):
        pltpu.make_async_copy(
            ws_hbm.at[pl.ds(0, G * SUB)], t_v[b], sem_g[b]).wait()

    def drain_out(b):
        for g in range(G):
            pltpu.make_async_copy(
                out_hbm.at[0, :, 0],
                tr_v[b].at[pl.ds(g * EMBED_DIM, EMBED_DIM), pl.ds(0, SUB)],
                sem_o[b]).wait()

    fire(0, 0)
    fire(1, 1)

    def super_step(ss, _):
        for b in range(2):
            c = ss * 2 + b
            drain_gathers(b)

            @pl.when(ss >= 1)
            def _drain():
                drain_out(b)

            for g in range(G):
                ridx0 = lane_iota + g * EMBED_DIM
                ridx1 = ridx0 + LANES

                def add_row(p, cidx, g=g, b=b, ridx0=ridx0, ridx1=ridx1):
                    x0 = t_v[b][g * SUB + p, pl.ds(0, LANES)]
                    plsc.store_scatter(tr_v[b], [ridx0, cidx], x0)
                    x1 = t_v[b][g * SUB + p, pl.ds(LANES, LANES)]
                    plsc.store_scatter(tr_v[b], [ridx1, cidx], x1)
                    return cidx + 1

                lax.fori_loop(0, SUB, add_row,
                              jnp.zeros((LANES,), jnp.int32), unroll=8)

            for g in range(G):
                grow = row0_w + c * G + g
                s = grow // N_BT
                bt = grow % N_BT
                for db in range(4):
                    pltpu.async_copy(
                        tr_v[b].at[pl.ds(g * EMBED_DIM + db * 8, 8),
                                   pl.ds(0, SUB)],
                        out_hbm.at[s, db, bt], sem_o[b])

            @pl.when(ss < N_CHUNKS // 2 - 1)
            def _prefetch():
                fire(c + 2, b)

        return _

    lax.fori_loop(0, N_CHUNKS // 2, super_step, 0)
    drain_out(0)
    drain_out(1)


_lookup = functools.partial(
    pl.kernel,
    out_type=jax.ShapeDtypeStruct((SEQ, 4, N_BT, 8, SUB), jnp.float32),
    mesh=_mesh,
    scratch_types=[
        pltpu.VMEM((G, SUB), jnp.int32),
        pltpu.VMEM((G, SUB), jnp.int32),
        pltpu.VMEM((G * SUB, EMBED_DIM), jnp.float32),
        pltpu.VMEM((G * SUB, EMBED_DIM), jnp.float32),
        pltpu.VMEM((G * EMBED_DIM, SUB + 1), jnp.float32),
        pltpu.VMEM((G * EMBED_DIM, SUB + 1), jnp.float32),
        pltpu.SemaphoreType.DMA,
        pltpu.SemaphoreType.DMA,
        pltpu.SemaphoreType.DMA,
        pltpu.SemaphoreType.DMA,
    ],
    compiler_params=pltpu.CompilerParams(
        use_tc_tiling_on_sc=False, needs_layout_passes=False),
)(_body)


@jax.jit
def kernel(input_ids, W_token, W_number):
    tail = (W_token[VOCAB - 64:] + W_number[VOCAB - 64:]).reshape(16, SUB)
    rs = _sumtable(W_token.T, W_number.T, tail)
    ws = rs.reshape(VOCAB, EMBED_DIM)

    ids = input_ids.astype(jnp.int32).T.reshape(N_GROUPS, SUB)
    out = _lookup(ids, ws)
    return out.transpose(2, 4, 0, 1, 3).reshape(BATCH, SEQ, EMBED_DIM)


# speedup vs baseline: 1.0564x; 1.0564x over previous
import functools

import jax
import jax.numpy as jnp
from jax import lax
from jax.experimental import pallas as pl
from jax.experimental.pallas import tpu as pltpu
from jax.experimental.pallas import tpu_sc as plsc

VOCAB = 1000000
EMBED_DIM = 32
BATCH = 4096
SEQ = 200

LANES = 16
SUB = 128
N_BT = BATCH // SUB
N_GROUPS = SEQ * N_BT
NW = 32
PER_W = N_GROUPS // NW
G = 4
N_CHUNKS = PER_W // G

N_WIN = (VOCAB - 64) // SUB
WIN_FLOOR = N_WIN // NW
WIN_EXTRA = N_WIN % NW
R_ROWS = VOCAB * EMBED_DIM // SUB

_mesh = plsc.VectorSubcoreMesh(core_axis_name="c", subcore_axis_name="s")


def _sum_body(wt_t, wn_t, tail, rs, at0, at1, an0, an1, o0, o1, tl_v,
              si0, si1, so0, so1):
    wid = lax.axis_index("s") * 2 + lax.axis_index("c")
    lane_iota = lax.iota(jnp.int32, LANES)

    at_v = (at0, at1)
    an_v = (an0, an1)
    o_v = (o0, o1)
    sem_i = (si0, si1)
    sem_o = (so0, so1)

    @pl.when(wid == 0)
    def _tail():
        pltpu.sync_copy(tail, tl_v)
        pltpu.sync_copy(tl_v, rs.at[pl.ds(R_ROWS - 16, 16)])

    def fire_in(jj, b):
        j = wid + NW * jj
        for db in range(4):
            pltpu.async_copy(
                wt_t.at[pl.ds(8 * db, 8), pl.ds(SUB * j, SUB)],
                at_v[b].at[db], sem_i[b])
            pltpu.async_copy(
                wn_t.at[pl.ds(8 * db, 8), pl.ds(SUB * j, SUB)],
                an_v[b].at[db], sem_i[b])

    def drain_in(b):
        for db in range(4):
            pltpu.make_async_copy(
                wt_t.at[pl.ds(0, 8), pl.ds(0, SUB)], at_v[b].at[db],
                sem_i[b]).wait()
            pltpu.make_async_copy(
                wn_t.at[pl.ds(0, 8), pl.ds(0, SUB)], an_v[b].at[db],
                sem_i[b]).wait()

    def drain_out(b):
        pltpu.make_async_copy(rs.at[pl.ds(0, 32)], o_v[b], sem_o[b]).wait()

    def sum_permute(b):
        def d_step(d0, _):
            dv = lax.rem(d0 + lane_iota, EMBED_DIM)
            dbv = dv // 8
            slv = dv - dbv * 8
            base = lane_iota * EMBED_DIM + dv
            qb = base // SUB
            cv2 = base - qb * SUB
            for k in range(8):
                cv = lane_iota + 16 * k
                x = (plsc.load_gather(at_v[b], [dbv, slv, cv]) +
                     plsc.load_gather(an_v[b], [dbv, slv, cv]))
                plsc.store_scatter(o_v[b], [qb + 4 * k, cv2], x)
            return _

        lax.fori_loop(0, EMBED_DIM, d_step, 0)

    fire_in(0, 0)
    fire_in(1, 1)

    def super_step(ss, _):
        for b in range(2):
            jj = ss * 2 + b
            drain_in(b)

            @pl.when(ss >= 1)
            def _do():
                drain_out(b)

            sum_permute(b)
            j = wid + NW * jj
            pltpu.async_copy(o_v[b], rs.at[pl.ds(32 * j, 32)], sem_o[b])

            @pl.when(ss < WIN_FLOOR // 2 - 1)
            def _pf():
                fire_in(jj + 2, b)

        return _

    lax.fori_loop(0, WIN_FLOOR // 2, super_step, 0)

    @pl.when(wid < WIN_EXTRA)
    def _extra():
        drain_out(0)
        j = wid + NW * WIN_FLOOR
        for db in range(4):
            pltpu.sync_copy(
                wt_t.at[pl.ds(8 * db, 8), pl.ds(SUB * j, SUB)],
                at_v[0].at[db])
            pltpu.sync_copy(
                wn_t.at[pl.ds(8 * db, 8), pl.ds(SUB * j, SUB)],
                an_v[0].at[db])
        sum_permute(0)
        pltpu.async_copy(o_v[0], rs.at[pl.ds(32 * j, 32)], sem_o[0])

    drain_out(0)
    drain_out(1)


_sumtable = functools.partial(
    pl.kernel,
    out_type=jax.ShapeDtypeStruct((R_ROWS, SUB), jnp.float32),
    mesh=_mesh,
    scratch_types=[
        pltpu.VMEM((4, 8, SUB), jnp.float32),
        pltpu.VMEM((4, 8, SUB), jnp.float32),
        pltpu.VMEM((4, 8, SUB), jnp.float32),
        pltpu.VMEM((4, 8, SUB), jnp.float32),
        pltpu.VMEM((32, SUB), jnp.float32),
        pltpu.VMEM((32, SUB), jnp.float32),
        pltpu.VMEM((16, SUB), jnp.float32),
        pltpu.SemaphoreType.DMA,
        pltpu.SemaphoreType.DMA,
        pltpu.SemaphoreType.DMA,
        pltpu.SemaphoreType.DMA,
    ],
    compiler_params=pltpu.CompilerParams(
        use_tc_tiling_on_sc=True, needs_layout_passes=False),
)(_sum_body)


def _body(ids_hbm, ws_hbm, out_hbm,
          idx0, idx1, t0, t1, tr0, tr1,
          sg0, sg1, so0, so1, sx0, sx1):
    wid = lax.axis_index("s") * 2 + lax.axis_index("c")
    row0_w = wid * PER_W
    lane_iota = lax.iota(jnp.int32, LANES)

    idx_v = (idx0, idx1)
    t_v = (t0, t1)
    tr_v = (tr0, tr1)
    sem_g = (sg0, sg1)
    sem_o = (so0, so1)
    sem_x = (sx0, sx1)

    def stage_idx(c, b):
        pltpu.async_copy(ids_hbm.at[pl.ds(row0_w + c * G, G)], idx_v[b],
                         sem_x[b])

    def fire(c, b):
        pltpu.make_async_copy(ids_hbm.at[pl.ds(0, G)], idx_v[b],
                              sem_x[b]).wait()
        for g in range(G):
            pltpu.async_copy(
                ws_hbm.at[idx_v[b].at[g]], t_v[b].at[pl.ds(g * SUB, SUB)],
                sem_g[b])

    def drain_gathers(b):
        pltpu.make_async_copy(
            ws_hbm.at[pl.ds(0, G * SUB)], t_v[b], sem_g[b]).wait()

    def drain_out(b):
        for g in range(G):
            pltpu.make_async_copy(
                out_hbm.at[0, :, 0],
                tr_v[b].at[pl.ds(g * EMBED_DIM, EMBED_DIM), pl.ds(0, SUB)],
                sem_o[b]).wait()

    stage_idx(0, 0)
    stage_idx(1, 1)
    fire(0, 0)
    fire(1, 1)

    def super_step(ss, _):
        for b in range(2):
            c = ss * 2 + b
            drain_gathers(b)

            @pl.when(ss < N_CHUNKS // 2 - 1)
            def _stage():
                stage_idx(c + 2, b)

            @pl.when(ss >= 1)
            def _drain():
                drain_out(b)

            for g in range(G):
                ridx0 = lane_iota + g * EMBED_DIM
                ridx1 = ridx0 + LANES

                def add_row(p, cidx, g=g, b=b, ridx0=ridx0, ridx1=ridx1):
                    x0 = t_v[b][g * SUB + p, pl.ds(0, LANES)]
                    plsc.store_scatter(tr_v[b], [ridx0, cidx], x0)
                    x1 = t_v[b][g * SUB + p, pl.ds(LANES, LANES)]
                    plsc.store_scatter(tr_v[b], [ridx1, cidx], x1)
                    return cidx + 1

                lax.fori_loop(0, SUB, add_row,
                              jnp.zeros((LANES,), jnp.int32), unroll=8)

            for g in range(G):
                grow = row0_w + c * G + g
                s = grow // N_BT
                bt = grow % N_BT
                for db in range(4):
                    pltpu.async_copy(
                        tr_v[b].at[pl.ds(g * EMBED_DIM + db * 8, 8),
                                   pl.ds(0, SUB)],
                        out_hbm.at[s, db, bt], sem_o[b])

            @pl.when(ss < N_CHUNKS // 2 - 1)
            def _prefetch():
                fire(c + 2, b)

        return _

    lax.fori_loop(0, N_CHUNKS // 2, super_step, 0)
    drain_out(0)
    drain_out(1)


_lookup = functools.partial(
    pl.kernel,
    out_type=jax.ShapeDtypeStruct((SEQ, 4, N_BT, 8, SUB), jnp.float32),
    mesh=_mesh,
    scratch_types=[
        pltpu.VMEM((G, SUB), jnp.int32),
        pltpu.VMEM((G, SUB), jnp.int32),
        pltpu.VMEM((G * SUB, EMBED_DIM), jnp.float32),
        pltpu.VMEM((G * SUB, EMBED_DIM), jnp.float32),
        pltpu.VMEM((G * EMBED_DIM, SUB + 1), jnp.float32),
        pltpu.VMEM((G * EMBED_DIM, SUB + 1), jnp.float32),
        pltpu.SemaphoreType.DMA,
        pltpu.SemaphoreType.DMA,
        pltpu.SemaphoreType.DMA,
        pltpu.SemaphoreType.DMA,
        pltpu.SemaphoreType.DMA,
        pltpu.SemaphoreType.DMA,
    ],
    compiler_params=pltpu.CompilerParams(
        use_tc_tiling_on_sc=False, needs_layout_passes=False),
)(_body)


@jax.jit
def kernel(input_ids, W_token, W_number):
    tail = (W_token[VOCAB - 64:] + W_number[VOCAB - 64:]).reshape(16, SUB)
    rs = _sumtable(W_token.T, W_number.T, tail)
    ws = rs.reshape(VOCAB, EMBED_DIM)

    ids = input_ids.astype(jnp.int32).T.reshape(N_GROUPS, SUB)
    out = _lookup(ids, ws)
    return out.transpose(2, 4, 0, 1, 3).reshape(BATCH, SEQ, EMBED_DIM)


# speedup vs baseline: 1.3274x; 1.2566x over previous
import functools

import jax
import jax.numpy as jnp
from jax import lax
from jax.experimental import pallas as pl
from jax.experimental.pallas import tpu as pltpu
from jax.experimental.pallas import tpu_sc as plsc

VOCAB = 1000000
EMBED_DIM = 32
BATCH = 4096
SEQ = 200

LANES = 16
SUB = 128
N_BT = BATCH // SUB
N_GROUPS = SEQ * N_BT
NW = 32
PER_W = N_GROUPS // NW
G = 4
N_CHUNKS = PER_W // G

N_WIN = (VOCAB - 64) // SUB
WIN_FLOOR = N_WIN // NW
WIN_EXTRA = N_WIN % NW
R_ROWS = VOCAB * EMBED_DIM // SUB

_mesh = plsc.VectorSubcoreMesh(core_axis_name="c", subcore_axis_name="s")


def _sum_body(wt_t, wn_t, tail, rs, at0, at1, an0, an1, o0, o1, tl_v,
              si0, si1, so0, so1):
    wid = lax.axis_index("s") * 2 + lax.axis_index("c")
    lane_iota = lax.iota(jnp.int32, LANES)

    at_v = (at0, at1)
    an_v = (an0, an1)
    o_v = (o0, o1)
    sem_i = (si0, si1)
    sem_o = (so0, so1)

    @pl.when(wid == 0)
    def _tail():
        pltpu.sync_copy(tail, tl_v)
        pltpu.sync_copy(tl_v, rs.at[pl.ds(R_ROWS - 16, 16)])

    def fire_in(jj, b):
        j = wid + NW * jj
        for db in range(4):
            pltpu.async_copy(
                wt_t.at[pl.ds(8 * db, 8), pl.ds(SUB * j, SUB)],
                at_v[b].at[db], sem_i[b])
            pltpu.async_copy(
                wn_t.at[pl.ds(8 * db, 8), pl.ds(SUB * j, SUB)],
                an_v[b].at[db], sem_i[b])

    def drain_in(b):
        for db in range(4):
            pltpu.make_async_copy(
                wt_t.at[pl.ds(0, 8), pl.ds(0, SUB)], at_v[b].at[db],
                sem_i[b]).wait()
            pltpu.make_async_copy(
                wn_t.at[pl.ds(0, 8), pl.ds(0, SUB)], an_v[b].at[db],
                sem_i[b]).wait()

    def drain_out(b):
        pltpu.make_async_copy(rs.at[pl.ds(0, 32)], o_v[b], sem_o[b]).wait()

    def sum_permute(b):
        def d_step(d0, _):
            dv = lax.rem(d0 + lane_iota, EMBED_DIM)
            dbv = dv // 8
            slv = dv - dbv * 8
            base = lane_iota * EMBED_DIM + dv
            qb = base // SUB
            cv2 = base - qb * SUB
            for k in range(8):
                cv = lane_iota + 16 * k
                x = (plsc.load_gather(at_v[b], [dbv, slv, cv]) +
                     plsc.load_gather(an_v[b], [dbv, slv, cv]))
                plsc.store_scatter(o_v[b], [qb + 4 * k, cv2], x)
            return _

        lax.fori_loop(0, EMBED_DIM, d_step, 0)

    fire_in(0, 0)
    fire_in(1, 1)

    def super_step(ss, _):
        for b in range(2):
            jj = ss * 2 + b
            drain_in(b)

            @pl.when(ss >= 1)
            def _do():
                drain_out(b)

            sum_permute(b)
            j = wid + NW * jj
            pltpu.async_copy(o_v[b], rs.at[pl.ds(32 * j, 32)], sem_o[b])

            @pl.when(ss < WIN_FLOOR // 2 - 1)
            def _pf():
                fire_in(jj + 2, b)

        return _

    lax.fori_loop(0, WIN_FLOOR // 2, super_step, 0)

    @pl.when(wid < WIN_EXTRA)
    def _extra():
        drain_out(0)
        j = wid + NW * WIN_FLOOR
        for db in range(4):
            pltpu.sync_copy(
                wt_t.at[pl.ds(8 * db, 8), pl.ds(SUB * j, SUB)],
                at_v[0].at[db])
            pltpu.sync_copy(
                wn_t.at[pl.ds(8 * db, 8), pl.ds(SUB * j, SUB)],
                an_v[0].at[db])
        sum_permute(0)
        pltpu.async_copy(o_v[0], rs.at[pl.ds(32 * j, 32)], sem_o[0])

    drain_out(0)
    drain_out(1)


_sumtable = functools.partial(
    pl.kernel,
    out_type=jax.ShapeDtypeStruct((R_ROWS, SUB), jnp.float32),
    mesh=_mesh,
    scratch_types=[
        pltpu.VMEM((4, 8, SUB), jnp.float32),
        pltpu.VMEM((4, 8, SUB), jnp.float32),
        pltpu.VMEM((4, 8, SUB), jnp.float32),
        pltpu.VMEM((4, 8, SUB), jnp.float32),
        pltpu.VMEM((32, SUB), jnp.float32),
        pltpu.VMEM((32, SUB), jnp.float32),
        pltpu.VMEM((16, SUB), jnp.float32),
        pltpu.SemaphoreType.DMA,
        pltpu.SemaphoreType.DMA,
        pltpu.SemaphoreType.DMA,
        pltpu.SemaphoreType.DMA,
    ],
    compiler_params=pltpu.CompilerParams(
        use_tc_tiling_on_sc=True, needs_layout_passes=False),
)(_sum_body)


def _body(ids_hbm, ws_hbm, out_hbm,
          idx0, idx1, t0, t1, tr0, tr1,
          sg0, sg1, so0, so1, sx0, sx1):
    wid = lax.axis_index("s") * 2 + lax.axis_index("c")
    row0_w = wid * PER_W
    lane_iota = lax.iota(jnp.int32, LANES)

    idx_v = (idx0, idx1)
    t_v = (t0, t1)
    tr_v = (tr0, tr1)
    sem_g = (sg0, sg1)
    sem_o = (so0, so1)
    sem_x = (sx0, sx1)

    def stage_idx(c, b):
        pltpu.async_copy(ids_hbm.at[pl.ds(row0_w + c * G, G)], idx_v[b],
                         sem_x[b])

    def fire(c, b):
        pltpu.make_async_copy(ids_hbm.at[pl.ds(0, G)], idx_v[b],
                              sem_x[b]).wait()
        for g in range(G):
            pltpu.async_copy(
                ws_hbm.at[idx_v[b].at[g]], t_v[b].at[pl.ds(g * SUB, SUB)],
                sem_g[b])

    def drain_gathers(b):
        pltpu.make_async_copy(
            ws_hbm.at[pl.ds(0, G * SUB)], t_v[b], sem_g[b]).wait()

    def drain_out(b):
        for g in range(G):
            pltpu.make_async_copy(
                out_hbm.at[0, :, 0],
                tr_v[b].at[pl.ds(g * EMBED_DIM, EMBED_DIM), pl.ds(0, SUB)],
                sem_o[b]).wait()

    stage_idx(0, 0)
    stage_idx(1, 1)
    fire(0, 0)
    fire(1, 1)

    def super_step(ss, _):
        for b in range(2):
            c = ss * 2 + b
            drain_gathers(b)

            @pl.when(ss < N_CHUNKS // 2 - 1)
            def _stage():
                stage_idx(c + 2, b)

            @pl.when(ss >= 1)
            def _drain():
                drain_out(b)

            for g in range(G):
                ridx0 = lane_iota + g * EMBED_DIM
                ridx1 = ridx0 + LANES

                @plsc.parallel_loop(0, SUB, unroll=8,
                                    carry=jnp.zeros((LANES,), jnp.int32))
                def add_row(p, cidx, g=g, b=b, ridx0=ridx0, ridx1=ridx1):
                    x0 = t_v[b][g * SUB + p, pl.ds(0, LANES)]
                    plsc.store_scatter(tr_v[b], [ridx0, cidx], x0)
                    x1 = t_v[b][g * SUB + p, pl.ds(LANES, LANES)]
                    plsc.store_scatter(tr_v[b], [ridx1, cidx], x1)
                    return cidx + 1

            for g in range(G):
                grow = row0_w + c * G + g
                s = grow // N_BT
                bt = grow % N_BT
                for db in range(4):
                    pltpu.async_copy(
                        tr_v[b].at[pl.ds(g * EMBED_DIM + db * 8, 8),
                                   pl.ds(0, SUB)],
                        out_hbm.at[s, db, bt], sem_o[b])

            @pl.when(ss < N_CHUNKS // 2 - 1)
            def _prefetch():
                fire(c + 2, b)

        return _

    lax.fori_loop(0, N_CHUNKS // 2, super_step, 0)
    drain_out(0)
    drain_out(1)


_lookup = functools.partial(
    pl.kernel,
    out_type=jax.ShapeDtypeStruct((SEQ, 4, N_BT, 8, SUB), jnp.float32),
    mesh=_mesh,
    scratch_types=[
        pltpu.VMEM((G, SUB), jnp.int32),
        pltpu.VMEM((G, SUB), jnp.int32),
        pltpu.VMEM((G * SUB, EMBED_DIM), jnp.float32),
        pltpu.VMEM((G * SUB, EMBED_DIM), jnp.float32),
        pltpu.VMEM((G * EMBED_DIM, SUB + 1), jnp.float32),
        pltpu.VMEM((G * EMBED_DIM, SUB + 1), jnp.float32),
        pltpu.SemaphoreType.DMA,
        pltpu.SemaphoreType.DMA,
        pltpu.SemaphoreType.DMA,
        pltpu.SemaphoreType.DMA,
        pltpu.SemaphoreType.DMA,
        pltpu.SemaphoreType.DMA,
    ],
    compiler_params=pltpu.CompilerParams(
        use_tc_tiling_on_sc=False, needs_layout_passes=False),
)(_body)


@jax.jit
def kernel(input_ids, W_token, W_number):
    tail = (W_token[VOCAB - 64:] + W_number[VOCAB - 64:]).reshape(16, SUB)
    rs = _sumtable(W_token.T, W_number.T, tail)
    ws = rs.reshape(VOCAB, EMBED_DIM)

    ids = input_ids.astype(jnp.int32).T.reshape(N_GROUPS, SUB)
    out = _lookup(ids, ws)
    return out.transpose(2, 4, 0, 1, 3).reshape(BATCH, SEQ, EMBED_DIM)


# speedup vs baseline: 2.0816x; 1.5681x over previous
import functools

import jax
import jax.numpy as jnp
from jax import lax
from jax.experimental import pallas as pl
from jax.experimental.pallas import tpu as pltpu
from jax.experimental.pallas import tpu_sc as plsc

VOCAB = 1000000
EMBED_DIM = 32
BATCH = 4096
SEQ = 200

LANES = 16
SUB = 128
N_BT = BATCH // SUB
N_GROUPS = SEQ * N_BT
NW = 32
PER_W = N_GROUPS // NW
G = 4
N_CHUNKS = PER_W // G

N_WIN = (VOCAB - 64) // SUB
WIN_FLOOR = N_WIN // NW
WIN_EXTRA = N_WIN % NW
R_ROWS = VOCAB * EMBED_DIM // SUB

_mesh = plsc.VectorSubcoreMesh(core_axis_name="c", subcore_axis_name="s")


def _sum_body(wt_t, wn_t, tail, rs, at0, at1, an0, an1, o0, o1, tl_v,
              si0, si1, so0, so1):
    wid = lax.axis_index("s") * 2 + lax.axis_index("c")
    lane_iota = lax.iota(jnp.int32, LANES)

    at_v = (at0, at1)
    an_v = (an0, an1)
    o_v = (o0, o1)
    sem_i = (si0, si1)
    sem_o = (so0, so1)

    @pl.when(wid == 0)
    def _tail():
        pltpu.sync_copy(tail, tl_v)
        pltpu.sync_copy(tl_v, rs.at[pl.ds(R_ROWS - 16, 16)])

    def fire_in(jj, b):
        j = wid + NW * jj
        for db in range(4):
            pltpu.async_copy(
                wt_t.at[pl.ds(8 * db, 8), pl.ds(SUB * j, SUB)],
                at_v[b].at[db], sem_i[b])
            pltpu.async_copy(
                wn_t.at[pl.ds(8 * db, 8), pl.ds(SUB * j, SUB)],
                an_v[b].at[db], sem_i[b])

    def drain_in(b):
        for db in range(4):
            pltpu.make_async_copy(
                wt_t.at[pl.ds(0, 8), pl.ds(0, SUB)], at_v[b].at[db],
                sem_i[b]).wait()
            pltpu.make_async_copy(
                wn_t.at[pl.ds(0, 8), pl.ds(0, SUB)], an_v[b].at[db],
                sem_i[b]).wait()

    def drain_out(b):
        pltpu.make_async_copy(rs.at[pl.ds(0, 32)], o_v[b], sem_o[b]).wait()

    def sum_permute(b):
        @plsc.parallel_loop(0, EMBED_DIM, unroll=4)
        def d_step(d0, b=b):
            dv = lax.rem(d0 + lane_iota, EMBED_DIM)
            dbv = dv // 8
            slv = dv - dbv * 8
            base = lane_iota * EMBED_DIM + dv
            qb = base // SUB
            cv2 = base - qb * SUB
            for k in range(8):
                cv = lane_iota + 16 * k
                x = (plsc.load_gather(at_v[b], [dbv, slv, cv]) +
                     plsc.load_gather(an_v[b], [dbv, slv, cv]))
                plsc.store_scatter(o_v[b], [qb + 4 * k, cv2], x)

    fire_in(0, 0)
    fire_in(1, 1)

    def super_step(ss, _):
        for b in range(2):
            jj = ss * 2 + b
            drain_in(b)

            @pl.when(ss >= 1)
            def _do():
                drain_out(b)

            sum_permute(b)
            j = wid + NW * jj
            pltpu.async_copy(o_v[b], rs.at[pl.ds(32 * j, 32)], sem_o[b])

            @pl.when(ss < WIN_FLOOR // 2 - 1)
            def _pf():
                fire_in(jj + 2, b)

        return _

    lax.fori_loop(0, WIN_FLOOR // 2, super_step, 0)

    @pl.when(wid < WIN_EXTRA)
    def _extra():
        drain_out(0)
        j = wid + NW * WIN_FLOOR
        for db in range(4):
            pltpu.sync_copy(
                wt_t.at[pl.ds(8 * db, 8), pl.ds(SUB * j, SUB)],
                at_v[0].at[db])
            pltpu.sync_copy(
                wn_t.at[pl.ds(8 * db, 8), pl.ds(SUB * j, SUB)],
                an_v[0].at[db])
        sum_permute(0)
        pltpu.async_copy(o_v[0], rs.at[pl.ds(32 * j, 32)], sem_o[0])

    drain_out(0)
    drain_out(1)


_sumtable = functools.partial(
    pl.kernel,
    out_type=jax.ShapeDtypeStruct((R_ROWS, SUB), jnp.float32),
    mesh=_mesh,
    scratch_types=[
        pltpu.VMEM((4, 8, SUB), jnp.float32),
        pltpu.VMEM((4, 8, SUB), jnp.float32),
        pltpu.VMEM((4, 8, SUB), jnp.float32),
        pltpu.VMEM((4, 8, SUB), jnp.float32),
        pltpu.VMEM((32, SUB), jnp.float32),
        pltpu.VMEM((32, SUB), jnp.float32),
        pltpu.VMEM((16, SUB), jnp.float32),
        pltpu.SemaphoreType.DMA,
        pltpu.SemaphoreType.DMA,
        pltpu.SemaphoreType.DMA,
        pltpu.SemaphoreType.DMA,
    ],
    compiler_params=pltpu.CompilerParams(
        use_tc_tiling_on_sc=True, needs_layout_passes=False),
)(_sum_body)


def _body(ids_hbm, ws_hbm, out_hbm,
          idx0, idx1, t0, t1, tr0, tr1,
          sg0, sg1, so0, so1, sx0, sx1):
    wid = lax.axis_index("s") * 2 + lax.axis_index("c")
    row0_w = wid * PER_W
    lane_iota = lax.iota(jnp.int32, LANES)

    idx_v = (idx0, idx1)
    t_v = (t0, t1)
    tr_v = (tr0, tr1)
    sem_g = (sg0, sg1)
    sem_o = (so0, so1)
    sem_x = (sx0, sx1)

    def stage_idx(c, b):
        pltpu.async_copy(ids_hbm.at[pl.ds(row0_w + c * G, G)], idx_v[b],
                         sem_x[b])

    def fire(c, b):
        pltpu.make_async_copy(ids_hbm.at[pl.ds(0, G)], idx_v[b],
                              sem_x[b]).wait()
        for g in range(G):
            pltpu.async_copy(
                ws_hbm.at[idx_v[b].at[g]], t_v[b].at[pl.ds(g * SUB, SUB)],
                sem_g[b])

    def drain_gathers(b):
        pltpu.make_async_copy(
            ws_hbm.at[pl.ds(0, G * SUB)], t_v[b], sem_g[b]).wait()

    def drain_out(b):
        for g in range(G):
            pltpu.make_async_copy(
                out_hbm.at[0, :, 0],
                tr_v[b].at[pl.ds(g * EMBED_DIM, EMBED_DIM), pl.ds(0, SUB)],
                sem_o[b]).wait()

    stage_idx(0, 0)
    stage_idx(1, 1)
    fire(0, 0)
    fire(1, 1)

    def super_step(ss, _):
        for b in range(2):
            c = ss * 2 + b
            drain_gathers(b)

            @pl.when(ss < N_CHUNKS // 2 - 1)
            def _stage():
                stage_idx(c + 2, b)

            @pl.when(ss >= 1)
            def _drain():
                drain_out(b)

            for g in range(G):
                ridx0 = lane_iota + g * EMBED_DIM
                ridx1 = ridx0 + LANES

                @plsc.parallel_loop(0, SUB, unroll=8,
                                    carry=jnp.zeros((LANES,), jnp.int32))
                def add_row(p, cidx, g=g, b=b, ridx0=ridx0, ridx1=ridx1):
                    x0 = t_v[b][g * SUB + p, pl.ds(0, LANES)]
                    plsc.store_scatter(tr_v[b], [ridx0, cidx], x0)
                    x1 = t_v[b][g * SUB + p, pl.ds(LANES, LANES)]
                    plsc.store_scatter(tr_v[b], [ridx1, cidx], x1)
                    return cidx + 1

            for g in range(G):
                grow = row0_w + c * G + g
                s = grow // N_BT
                bt = grow % N_BT
                for db in range(4):
                    pltpu.async_copy(
                        tr_v[b].at[pl.ds(g * EMBED_DIM + db * 8, 8),
                                   pl.ds(0, SUB)],
                        out_hbm.at[s, db, bt], sem_o[b])

            @pl.when(ss < N_CHUNKS // 2 - 1)
            def _prefetch():
                fire(c + 2, b)

        return _

    lax.fori_loop(0, N_CHUNKS // 2, super_step, 0)
    drain_out(0)
    drain_out(1)


_lookup = functools.partial(
    pl.kernel,
    out_type=jax.ShapeDtypeStruct((SEQ, 4, N_BT, 8, SUB), jnp.float32),
    mesh=_mesh,
    scratch_types=[
        pltpu.VMEM((G, SUB), jnp.int32),
        pltpu.VMEM((G, SUB), jnp.int32),
        pltpu.VMEM((G * SUB, EMBED_DIM), jnp.float32),
        pltpu.VMEM((G * SUB, EMBED_DIM), jnp.float32),
        pltpu.VMEM((G * EMBED_DIM, SUB + 1), jnp.float32),
        pltpu.VMEM((G * EMBED_DIM, SUB + 1), jnp.float32),
        pltpu.SemaphoreType.DMA,
        pltpu.SemaphoreType.DMA,
        pltpu.SemaphoreType.DMA,
        pltpu.SemaphoreType.DMA,
        pltpu.SemaphoreType.DMA,
        pltpu.SemaphoreType.DMA,
    ],
    compiler_params=pltpu.CompilerParams(
        use_tc_tiling_on_sc=False, needs_layout_passes=False),
)(_body)


@jax.jit
def kernel(input_ids, W_token, W_number):
    tail = (W_token[VOCAB - 64:] + W_number[VOCAB - 64:]).reshape(16, SUB)
    rs = _sumtable(W_token.T, W_number.T, tail)
    ws = rs.reshape(VOCAB, EMBED_DIM)

    ids = input_ids.astype(jnp.int32).T.reshape(N_GROUPS, SUB)
    out = _lookup(ids, ws)
    return out.transpose(2, 4, 0, 1, 3).reshape(BATCH, SEQ, EMBED_DIM)


# speedup vs baseline: 2.0879x; 1.0030x over previous
import functools

import jax
import jax.numpy as jnp
from jax import lax
from jax.experimental import pallas as pl
from jax.experimental.pallas import tpu as pltpu
from jax.experimental.pallas import tpu_sc as plsc

VOCAB = 1000000
EMBED_DIM = 32
BATCH = 4096
SEQ = 200

LANES = 16
SUB = 128
N_BT = BATCH // SUB
N_GROUPS = SEQ * N_BT
NW = 32
PER_W = N_GROUPS // NW
G = 5
N_CHUNKS = PER_W // G

N_WIN = (VOCAB - 64) // SUB
WIN_FLOOR = N_WIN // NW
WIN_EXTRA = N_WIN % NW
R_ROWS = VOCAB * EMBED_DIM // SUB

_mesh = plsc.VectorSubcoreMesh(core_axis_name="c", subcore_axis_name="s")


def _sum_body(wt_t, wn_t, tail, rs, at0, at1, an0, an1, o0, o1, tl_v,
              si0, si1, so0, so1):
    wid = lax.axis_index("s") * 2 + lax.axis_index("c")
    lane_iota = lax.iota(jnp.int32, LANES)

    at_v = (at0, at1)
    an_v = (an0, an1)
    o_v = (o0, o1)
    sem_i = (si0, si1)
    sem_o = (so0, so1)

    @pl.when(wid == 0)
    def _tail():
        pltpu.sync_copy(tail, tl_v)
        pltpu.sync_copy(tl_v, rs.at[pl.ds(R_ROWS - 16, 16)])

    def fire_in(jj, b):
        j = wid + NW * jj
        for db in range(4):
            pltpu.async_copy(
                wt_t.at[pl.ds(8 * db, 8), pl.ds(SUB * j, SUB)],
                at_v[b].at[db], sem_i[b])
            pltpu.async_copy(
                wn_t.at[pl.ds(8 * db, 8), pl.ds(SUB * j, SUB)],
                an_v[b].at[db], sem_i[b])

    def drain_in(b):
        for db in range(4):
            pltpu.make_async_copy(
                wt_t.at[pl.ds(0, 8), pl.ds(0, SUB)], at_v[b].at[db],
                sem_i[b]).wait()
            pltpu.make_async_copy(
                wn_t.at[pl.ds(0, 8), pl.ds(0, SUB)], an_v[b].at[db],
                sem_i[b]).wait()

    def drain_out(b):
        pltpu.make_async_copy(rs.at[pl.ds(0, 32)], o_v[b], sem_o[b]).wait()

    def sum_permute(b):
        @plsc.parallel_loop(0, EMBED_DIM, unroll=4)
        def d_step(d0, b=b):
            dv = lax.rem(d0 + lane_iota, EMBED_DIM)
            dbv = dv // 8
            slv = dv - dbv * 8
            base = lane_iota * EMBED_DIM + dv
            qb = base // SUB
            cv2 = base - qb * SUB
            for k in range(8):
                cv = lane_iota + 16 * k
                x = (plsc.load_gather(at_v[b], [dbv, slv, cv]) +
                     plsc.load_gather(an_v[b], [dbv, slv, cv]))
                plsc.store_scatter(o_v[b], [qb + 4 * k, cv2], x)

    fire_in(0, 0)
    fire_in(1, 1)

    def super_step(ss, _):
        for b in range(2):
            jj = ss * 2 + b
            drain_in(b)

            @pl.when(ss >= 1)
            def _do():
                drain_out(b)

            sum_permute(b)
            j = wid + NW * jj
            pltpu.async_copy(o_v[b], rs.at[pl.ds(32 * j, 32)], sem_o[b])

            @pl.when(ss < WIN_FLOOR // 2 - 1)
            def _pf():
                fire_in(jj + 2, b)

        return _

    lax.fori_loop(0, WIN_FLOOR // 2, super_step, 0)

    @pl.when(wid < WIN_EXTRA)
    def _extra():
        drain_out(0)
        j = wid + NW * WIN_FLOOR
        for db in range(4):
            pltpu.sync_copy(
                wt_t.at[pl.ds(8 * db, 8), pl.ds(SUB * j, SUB)],
                at_v[0].at[db])
            pltpu.sync_copy(
                wn_t.at[pl.ds(8 * db, 8), pl.ds(SUB * j, SUB)],
                an_v[0].at[db])
        sum_permute(0)
        pltpu.async_copy(o_v[0], rs.at[pl.ds(32 * j, 32)], sem_o[0])

    drain_out(0)
    drain_out(1)


_sumtable = functools.partial(
    pl.kernel,
    out_type=jax.ShapeDtypeStruct((R_ROWS, SUB), jnp.float32),
    mesh=_mesh,
    scratch_types=[
        pltpu.VMEM((4, 8, SUB), jnp.float32),
        pltpu.VMEM((4, 8, SUB), jnp.float32),
        pltpu.VMEM((4, 8, SUB), jnp.float32),
        pltpu.VMEM((4, 8, SUB), jnp.float32),
        pltpu.VMEM((32, SUB), jnp.float32),
        pltpu.VMEM((32, SUB), jnp.float32),
        pltpu.VMEM((16, SUB), jnp.float32),
        pltpu.SemaphoreType.DMA,
        pltpu.SemaphoreType.DMA,
        pltpu.SemaphoreType.DMA,
        pltpu.SemaphoreType.DMA,
    ],
    compiler_params=pltpu.CompilerParams(
        use_tc_tiling_on_sc=True, needs_layout_passes=False),
)(_sum_body)


def _body(ids_hbm, ws_hbm, out_hbm,
          idx0, idx1, t0, t1, tr0, tr1,
          sg0, sg1, so0, so1, sx0, sx1):
    wid = lax.axis_index("s") * 2 + lax.axis_index("c")
    row0_w = wid * PER_W
    lane_iota = lax.iota(jnp.int32, LANES)

    idx_v = (idx0, idx1)
    t_v = (t0, t1)
    tr_v = (tr0, tr1)
    sem_g = (sg0, sg1)
    sem_o = (so0, so1)
    sem_x = (sx0, sx1)

    def stage_idx(c, b):
        pltpu.async_copy(ids_hbm.at[pl.ds(row0_w + c * G, G)], idx_v[b],
                         sem_x[b])

    def fire(c, b):
        pltpu.make_async_copy(ids_hbm.at[pl.ds(0, G)], idx_v[b],
                              sem_x[b]).wait()
        for g in range(G):
            pltpu.async_copy(
                ws_hbm.at[idx_v[b].at[g]], t_v[b].at[pl.ds(g * SUB, SUB)],
                sem_g[b])

    def drain_gathers(b):
        pltpu.make_async_copy(
            ws_hbm.at[pl.ds(0, G * SUB)], t_v[b], sem_g[b]).wait()

    def drain_out(b):
        for g in range(G):
            pltpu.make_async_copy(
                out_hbm.at[0, :, 0],
                tr_v[b].at[pl.ds(g * EMBED_DIM, EMBED_DIM), pl.ds(0, SUB)],
                sem_o[b]).wait()

    stage_idx(0, 0)
    stage_idx(1, 1)
    fire(0, 0)
    fire(1, 1)

    def super_step(ss, _):
        for b in range(2):
            c = ss * 2 + b
            drain_gathers(b)

            @pl.when(ss < N_CHUNKS // 2 - 1)
            def _stage():
                stage_idx(c + 2, b)

            @pl.when(ss >= 1)
            def _drain():
                drain_out(b)

            for g in range(G):
                ridx0 = lane_iota + g * EMBED_DIM
                ridx1 = ridx0 + LANES

                @plsc.parallel_loop(0, SUB, unroll=8,
                                    carry=jnp.zeros((LANES,), jnp.int32))
                def add_row(p, cidx, g=g, b=b, ridx0=ridx0, ridx1=ridx1):
                    x0 = t_v[b][g * SUB + p, pl.ds(0, LANES)]
                    plsc.store_scatter(tr_v[b], [ridx0, cidx], x0)
                    x1 = t_v[b][g * SUB + p, pl.ds(LANES, LANES)]
                    plsc.store_scatter(tr_v[b], [ridx1, cidx], x1)
                    return cidx + 1

            for g in range(G):
                grow = row0_w + c * G + g
                s = grow // N_BT
                bt = grow % N_BT
                for db in range(4):
                    pltpu.async_copy(
                        tr_v[b].at[pl.ds(g * EMBED_DIM + db * 8, 8),
                                   pl.ds(0, SUB)],
                        out_hbm.at[s, db, bt], sem_o[b])

            @pl.when(ss < N_CHUNKS // 2 - 1)
            def _prefetch():
                fire(c + 2, b)

        return _

    lax.fori_loop(0, N_CHUNKS // 2, super_step, 0)
    drain_out(0)
    drain_out(1)


_lookup = functools.partial(
    pl.kernel,
    out_type=jax.ShapeDtypeStruct((SEQ, 4, N_BT, 8, SUB), jnp.float32),
    mesh=_mesh,
    scratch_types=[
        pltpu.VMEM((G, SUB), jnp.int32),
        pltpu.VMEM((G, SUB), jnp.int32),
        pltpu.VMEM((G * SUB, EMBED_DIM), jnp.float32),
        pltpu.VMEM((G * SUB, EMBED_DIM), jnp.float32),
        pltpu.VMEM((G * EMBED_DIM, SUB + 1), jnp.float32),
        pltpu.VMEM((G * EMBED_DIM, SUB + 1), jnp.float32),
        pltpu.SemaphoreType.DMA,
        pltpu.SemaphoreType.DMA,
        pltpu.SemaphoreType.DMA,
        pltpu.SemaphoreType.DMA,
        pltpu.SemaphoreType.DMA,
        pltpu.SemaphoreType.DMA,
    ],
    compiler_params=pltpu.CompilerParams(
        use_tc_tiling_on_sc=False, needs_layout_passes=False),
)(_body)


@jax.jit
def kernel(input_ids, W_token, W_number):
    tail = (W_token[VOCAB - 64:] + W_number[VOCAB - 64:]).reshape(16, SUB)
    rs = _sumtable(W_token.T, W_number.T, tail)
    ws = rs.reshape(VOCAB, EMBED_DIM)

    ids = input_ids.astype(jnp.int32).T.reshape(N_GROUPS, SUB)
    out = _lookup(ids, ws)
    return out.transpose(2, 4, 0, 1, 3).reshape(BATCH, SEQ, EMBED_DIM)
